# baseline (speedup 1.0000x reference)
import functools

import jax
import jax.numpy as jnp
from jax import lax
from jax.experimental import pallas as pl
from jax.experimental.pallas import tpu as pltpu
from jax.experimental.pallas import tpu_sc as plsc

N = 320000
NSEG = 10000
NSEG_PAD = 10240
D = 128
BN_EPS = 1e-5

NC = 2
NS = 16
NW = NC * NS
ROWS_PER_W = N // NW
CH = 80
NCH = ROWS_PER_W // CH
SEG_SLICE = NSEG_PAD // NS

_mesh = plsc.VectorSubcoreMesh(core_axis_name="c", subcore_axis_name="s")


@functools.partial(
    pl.kernel,
    out_type=jax.ShapeDtypeStruct((NC, NSEG_PAD, D), jnp.float32),
    mesh=_mesh,
    scratch_types=[
        pltpu.VMEM((NCH, CH), jnp.int32),
        pltpu.VMEM((CH, D), jnp.float32),
        pltpu.VMEM((CH, D), jnp.float32),
        pltpu.VMEM((CH, D), jnp.float32),
        pltpu.VMEM_SHARED((NSEG_PAD, D), jnp.float32),
        pltpu.SemaphoreType.DMA,
        pltpu.SemaphoreType.DMA,
        pltpu.SemaphoreType.DMA,
        pltpu.SemaphoreType.DMA,
        pltpu.SemaphoreType.DMA,
        pltpu.SemaphoreType.DMA,
    ],
)
def _segment_sum_sc(x_hbm, batch_hbm, zeros_hbm, out_hbm,
                    idx_v, x0, x1, x2, table_sh,
                    gs0, gs1, gs2, ss0, ss1, ss2):
    c = lax.axis_index("c")
    s = lax.axis_index("s")
    wid = c * NS + s
    base = wid * ROWS_PER_W
    bufs = (x0, x1, x2)
    gsems = (gs0, gs1, gs2)
    ssems = (ss0, ss1, ss2)

    def chunk(j):
        return x_hbm.at[pl.ds(base + j * CH, CH)]

    pltpu.sync_copy(batch_hbm.at[wid], idx_v)
    pltpu.sync_copy(zeros_hbm, table_sh.at[pl.ds(s * SEG_SLICE, SEG_SLICE)])
    plsc.subcore_barrier()

    pltpu.async_copy(chunk(0), bufs[0], gsems[0])
    pltpu.async_copy(chunk(1), bufs[1], gsems[1])

    def body(j3, carry):
        for k in range(3):
            j = 3 * j3 + k
            k2 = (k + 2) % 3
            pltpu.make_async_copy(chunk(j), bufs[k], gsems[k]).wait()
            pltpu.async_copy(bufs[k], table_sh.at[idx_v.at[j]], ssems[k],
                             add=True)
            if k == 0:
                @pl.when(j3 > 0)
                def _():
                    pltpu.make_async_copy(
                        bufs[k2], table_sh.at[idx_v.at[j]], ssems[k2]).wait()
            else:
                pltpu.make_async_copy(
                    bufs[k2], table_sh.at[idx_v.at[j]], ssems[k2]).wait()
            pltpu.async_copy(chunk(j + 2), bufs[k2], gsems[k2])
        return carry

    lax.fori_loop(0, (NCH - 2) // 3, body, 0)
    pltpu.make_async_copy(chunk(NCH - 2), bufs[0], gsems[0]).wait()
    pltpu.make_async_copy(bufs[2], table_sh.at[idx_v.at[NCH - 3]], ssems[2]).wait()
    sc123 = pltpu.async_copy(bufs[0], table_sh.at[idx_v.at[NCH - 2]], ssems[0],
                             add=True)
    pltpu.make_async_copy(chunk(NCH - 1), bufs[1], gsems[1]).wait()
    sc123.wait()
    pltpu.sync_copy(bufs[1], table_sh.at[idx_v.at[NCH - 1]], add=True)
    plsc.subcore_barrier()
    pltpu.sync_copy(
        table_sh.at[pl.ds(s * SEG_SLICE, SEG_SLICE)],
        out_hbm.at[c, pl.ds(s * SEG_SLICE, SEG_SLICE)],
    )


def _bn_body(p_ref, wlin_ref, gamma_ref, beta_ref, out_ref):
    summ = p_ref[0] + p_ref[1]
    summ = lax.dot_general(
        summ, wlin_ref[...], (((1,), (1,)), ((), ())),
        preferred_element_type=jnp.float32,
    )
    mean = jnp.sum(summ, axis=0, keepdims=True) / NSEG
    cent = summ - mean
    ssq = jnp.sum(cent * cent, axis=0, keepdims=True) - (
        (NSEG_PAD - NSEG) * mean * mean
    )
    var = ssq / NSEG
    y = cent / jnp.sqrt(var + BN_EPS) * gamma_ref[...] + beta_ref[...]
    out_ref[...] = jnp.maximum(y, 0.0)


_bn_call = pl.pallas_call(
    _bn_body,
    out_shape=jax.ShapeDtypeStruct((NSEG_PAD, D), jnp.float32),
)


@functools.partial(
    pl.kernel,
    out_type=jax.ShapeDtypeStruct((N, D), jnp.float32),
    mesh=_mesh,
    scratch_types=[
        pltpu.VMEM((NCH, CH), jnp.int32),
        pltpu.VMEM((CH, D), jnp.float32),
        pltpu.VMEM((CH, D), jnp.float32),
        pltpu.VMEM((CH, D), jnp.float32),
        pltpu.VMEM_SHARED((NSEG_PAD, D), jnp.float32),
        pltpu.SemaphoreType.DMA,
        pltpu.SemaphoreType.DMA,
        pltpu.SemaphoreType.DMA,
        pltpu.SemaphoreType.DMA,
        pltpu.SemaphoreType.DMA,
        pltpu.SemaphoreType.DMA,
    ],
)
def _gather_sc(table_hbm, batch_hbm, out_hbm,
               idx_v, g0, g1, g2, table_sh, gsa0, gsa1, gsa2, ws0, ws1, ws2):
    c = lax.axis_index("c")
    s = lax.axis_index("s")
    wid = c * NS + s
    base = wid * ROWS_PER_W
    bufs = (g0, g1, g2)
    gsems = (gsa0, gsa1, gsa2)
    wsems = (ws0, ws1, ws2)

    def outref(j):
        return out_hbm.at[pl.ds(base + j * CH, CH)]

    pltpu.sync_copy(
        table_hbm.at[pl.ds(s * SEG_SLICE, SEG_SLICE)],
        table_sh.at[pl.ds(s * SEG_SLICE, SEG_SLICE)],
    )
    pltpu.sync_copy(batch_hbm.at[wid], idx_v)
    plsc.subcore_barrier()

    pltpu.async_copy(table_sh.at[idx_v.at[0]], bufs[0], gsems[0])
    pltpu.async_copy(table_sh.at[idx_v.at[1]], bufs[1], gsems[1])

    def body(j3, carry):
        for k in range(3):
            j = 3 * j3 + k
            k2 = (k + 2) % 3
            pltpu.make_async_copy(
                table_sh.at[idx_v.at[j]], bufs[k], gsems[k]).wait()
            pltpu.async_copy(bufs[k], outref(j), wsems[k])
            if k == 0:
                @pl.when(j3 > 0)
                def _():
                    pltpu.make_async_copy(bufs[k2], outref(j), wsems[k2]).wait()
            else:
                pltpu.make_async_copy(bufs[k2], outref(j), wsems[k2]).wait()
            pltpu.async_copy(table_sh.at[idx_v.at[j + 2]], bufs[k2], gsems[k2])
        return carry

    lax.fori_loop(0, (NCH - 2) // 3, body, 0)
    pltpu.make_async_copy(
        table_sh.at[idx_v.at[NCH - 2]], bufs[0], gsems[0]).wait()
    pltpu.make_async_copy(bufs[2], outref(NCH - 3), wsems[2]).wait()
    w123 = pltpu.async_copy(bufs[0], outref(NCH - 2), wsems[0])
    pltpu.make_async_copy(
        table_sh.at[idx_v.at[NCH - 1]], bufs[1], gsems[1]).wait()
    w123.wait()
    pltpu.sync_copy(bufs[1], outref(NCH - 1))


_BR = 16000


def _mlp_body(x_ref, g_ref, w1_ref, w2_ref, out_ref):
    h = x_ref[...] + g_ref[...]
    h = lax.dot_general(
        h, w1_ref[...], (((1,), (1,)), ((), ())),
        preferred_element_type=jnp.float32,
    )
    h = jnp.maximum(h, 0.0)
    out_ref[...] = lax.dot_general(
        h, w2_ref[...], (((1,), (1,)), ((), ())),
        preferred_element_type=jnp.float32,
    )


_mlp_call = pl.pallas_call(
    _mlp_body,
    grid=(N // _BR,),
    in_specs=[
        pl.BlockSpec((_BR, D), lambda i: (i, 0)),
        pl.BlockSpec((_BR, D), lambda i: (i, 0)),
        pl.BlockSpec((D, D), lambda i: (0, 0)),
        pl.BlockSpec((D, D), lambda i: (0, 0)),
    ],
    out_specs=pl.BlockSpec((_BR, D), lambda i: (i, 0)),
    out_shape=jax.ShapeDtypeStruct((N, D), jnp.float32),
)


def kernel(x, edge_index, edge_attr, batch, W_lin, gamma, beta, W1, W2):
    del edge_index, edge_attr
    batch3 = batch.reshape(NW, NCH, CH)
    zeros = jnp.zeros((SEG_SLICE, D), jnp.float32)
    partials = _segment_sum_sc(x, batch3, zeros)
    table = _bn_call(partials, W_lin, gamma.reshape(1, D), beta.reshape(1, D))
    g = _gather_sc(table, batch3)
    return _mlp_call(x, g, W1, W2)
